# baseline (speedup 1.0000x reference)
import functools

import jax
import jax.numpy as jnp
import numpy as np
from jax import lax
from jax.experimental import pallas as pl
from jax.experimental.pallas import tpu as pltpu
from jax.experimental.pallas import tpu_sc as plsc

D_MODEL = 64
SEQ = 200
BATCH = 1024
CHUNK = 40
NCH = SEQ // CHUNK

_info = plsc.get_sparse_core_info()
NC, NS, LANES = _info.num_cores, _info.num_subcores, _info.num_lanes
NW = NC * NS
SEQ_PER_W = BATCH // NW


def _pos_encoding(seq_len: int, d_model: int) -> np.ndarray:
    pos = np.arange(seq_len)[:, np.newaxis].astype(np.float32)
    i = np.arange(d_model)[np.newaxis, :].astype(np.float32)
    angle_rates = 1.0 / np.power(
        10000.0, 2.0 * (np.floor(i / 2.0)) / np.float32(d_model))
    angle_rads = pos * angle_rates
    angle_rads[:, 0::2] = np.sin(angle_rads[:, 0::2])
    angle_rads[:, 1::2] = np.cos(angle_rads[:, 1::2])
    return angle_rads.astype(np.float32)


def _sc_body(table_hbm, x_hbm, pe_hbm, out_hbm, idx_v, pe_v, rows_v, sem):
    wid = lax.axis_index("s") * NC + lax.axis_index("c")
    pltpu.sync_copy(x_hbm.at[wid], idx_v)
    pltpu.sync_copy(pe_hbm, pe_v)
    base_row = wid * (SEQ_PER_W * SEQ)

    def seq_body(t, _):
        def chunk_body(c, _):
            pltpu.async_copy(table_hbm.at[idx_v.at[t, c]], rows_v, sem).wait()
            pe_off = c * CHUNK

            def row_body(r, _):
                for j in range(D_MODEL // LANES):
                    sl = pl.ds(j * LANES, LANES)
                    rows_v[r, sl] = rows_v[r, sl] * 8.0 + pe_v[pe_off + r, sl]
                return 0

            lax.fori_loop(0, CHUNK, row_body, 0, unroll=2)
            out_base = base_row + t * SEQ + pe_off
            pltpu.sync_copy(rows_v, out_hbm.at[pl.ds(out_base, CHUNK)])
            return 0

        lax.fori_loop(0, NCH, chunk_body, 0)
        return 0

    lax.fori_loop(0, SEQ_PER_W, seq_body, 0)


@jax.jit
def _run(table, xr, pe):
    mesh = plsc.VectorSubcoreMesh(core_axis_name="c", subcore_axis_name="s")
    f = functools.partial(
        pl.kernel,
        mesh=mesh,
        out_type=jax.ShapeDtypeStruct((BATCH * SEQ, D_MODEL), jnp.float32),
        scratch_types=[
            pltpu.VMEM((SEQ_PER_W, NCH, CHUNK), jnp.int32),
            pltpu.VMEM((SEQ, D_MODEL), jnp.float32),
            pltpu.VMEM((CHUNK, D_MODEL), jnp.float32),
            pltpu.SemaphoreType.DMA,
        ],
        compiler_params=pltpu.CompilerParams(use_tc_tiling_on_sc=False),
    )(_sc_body)
    return f(table, xr, pe)


def kernel(x, table):
    pe = jnp.asarray(_pos_encoding(SEQ, D_MODEL))
    xr = x.astype(jnp.int32).reshape(NW, SEQ_PER_W, NCH, CHUNK)
    out = _run(table, xr, pe)
    return out.reshape(BATCH, SEQ, D_MODEL)


# speedup vs baseline: 1.1387x; 1.1387x over previous
import functools

import jax
import jax.numpy as jnp
import numpy as np
from jax import lax
from jax.experimental import pallas as pl
from jax.experimental.pallas import tpu as pltpu
from jax.experimental.pallas import tpu_sc as plsc

D_MODEL = 64
SEQ = 200
BATCH = 1024
CHUNK = 40
NCH = SEQ // CHUNK

_info = plsc.get_sparse_core_info()
NC, NS, LANES = _info.num_cores, _info.num_subcores, _info.num_lanes
NW = NC * NS
SEQ_PER_W = BATCH // NW


def _pos_encoding(seq_len: int, d_model: int) -> np.ndarray:
    pos = np.arange(seq_len)[:, np.newaxis].astype(np.float32)
    i = np.arange(d_model)[np.newaxis, :].astype(np.float32)
    angle_rates = 1.0 / np.power(
        10000.0, 2.0 * (np.floor(i / 2.0)) / np.float32(d_model))
    angle_rads = pos * angle_rates
    angle_rads[:, 0::2] = np.sin(angle_rads[:, 0::2])
    angle_rads[:, 1::2] = np.cos(angle_rads[:, 1::2])
    return angle_rads.astype(np.float32)


NBUF = 8
LOOKAHEAD = 4
NCHW = SEQ_PER_W * NCH


def _sc_body(table_hbm, x_hbm, pe_hbm, out_hbm, idx_v, pe_v, rows_v,
             gsem, ssem):
    wid = lax.axis_index("s") * NC + lax.axis_index("c")
    pltpu.sync_copy(x_hbm.at[wid], idx_v)
    pltpu.sync_copy(pe_hbm, pe_v)
    base_row = wid * (SEQ_PER_W * SEQ)

    def fire_gather(c, b):
        pltpu.async_copy(table_hbm.at[idx_v.at[c]], rows_v.at[b], gsem)

    def wait_gather(c, b):
        pltpu.make_async_copy(
            table_hbm.at[idx_v.at[c]], rows_v.at[b], gsem).wait()

    def wait_scatter(b):
        pltpu.make_async_copy(
            rows_v.at[b], out_hbm.at[pl.ds(0, CHUNK)], ssem).wait()

    for b in range(LOOKAHEAD):
        fire_gather(b, b)

    def q_body(qi, _):
        q = qi * NBUF
        for b in range(NBUF):
            c = q + b
            ca = c + LOOKAHEAD
            ba = (b + LOOKAHEAD) % NBUF

            @pl.when(jnp.logical_and(ca >= NBUF, ca < NCHW))
            def _():
                wait_scatter(ba)
                fire_gather(ca, ba)

            @pl.when(jnp.logical_and(ca >= LOOKAHEAD, ca < NBUF))
            def _():
                fire_gather(ca, ba)

            wait_gather(c, b)
            m = lax.rem(c, NCH)

            def row_body(r, _):
                for j in range(D_MODEL // LANES):
                    sl = pl.ds(j * LANES, LANES)
                    rows_v[b, r, sl] = (
                        rows_v[b, r, sl] * 8.0 + pe_v[m, r, sl])
                return 0

            lax.fori_loop(0, CHUNK, row_body, 0, unroll=2)
            pltpu.async_copy(
                rows_v.at[b],
                out_hbm.at[pl.ds(base_row + c * CHUNK, CHUNK)], ssem)
        return 0

    lax.fori_loop(0, NCHW // NBUF, q_body, 0)
    for b in range(NBUF):
        wait_scatter(b)


@jax.jit
def _run(table, xr, pe):
    mesh = plsc.VectorSubcoreMesh(core_axis_name="c", subcore_axis_name="s")
    f = functools.partial(
        pl.kernel,
        mesh=mesh,
        out_type=jax.ShapeDtypeStruct((BATCH * SEQ, D_MODEL), jnp.float32),
        scratch_types=[
            pltpu.VMEM((NCHW, CHUNK), jnp.int32),
            pltpu.VMEM((NCH, CHUNK, D_MODEL), jnp.float32),
            pltpu.VMEM((NBUF, CHUNK, D_MODEL), jnp.float32),
            pltpu.SemaphoreType.DMA,
            pltpu.SemaphoreType.DMA,
        ],
        compiler_params=pltpu.CompilerParams(use_tc_tiling_on_sc=False),
    )(_sc_body)
    return f(table, xr, pe)


def kernel(x, table):
    pe = jnp.asarray(
        _pos_encoding(SEQ, D_MODEL).reshape(NCH, CHUNK, D_MODEL))
    xr = x.astype(jnp.int32).reshape(NW, NCHW, CHUNK)
    out = _run(table, xr, pe)
    return out.reshape(BATCH, SEQ, D_MODEL)


# speedup vs baseline: 1.1500x; 1.0099x over previous
import functools

import jax
import jax.numpy as jnp
import numpy as np
from jax import lax
from jax.experimental import pallas as pl
from jax.experimental.pallas import tpu as pltpu
from jax.experimental.pallas import tpu_sc as plsc

D_MODEL = 64
SEQ = 200
BATCH = 1024
CHUNK = 40
NCH = SEQ // CHUNK

_info = plsc.get_sparse_core_info()
NC, NS, LANES = _info.num_cores, _info.num_subcores, _info.num_lanes
NW = NC * NS
SEQ_PER_W = BATCH // NW

NBUF = 8
LOOKAHEAD = 4
NCHW = SEQ_PER_W * NCH


def _pos_encoding(seq_len: int, d_model: int) -> np.ndarray:
    pos = np.arange(seq_len)[:, np.newaxis].astype(np.float32)
    i = np.arange(d_model)[np.newaxis, :].astype(np.float32)
    angle_rates = 1.0 / np.power(
        10000.0, 2.0 * (np.floor(i / 2.0)) / np.float32(d_model))
    angle_rads = pos * angle_rates
    angle_rads[:, 0::2] = np.sin(angle_rads[:, 0::2])
    angle_rads[:, 1::2] = np.cos(angle_rads[:, 1::2])
    return angle_rads.astype(np.float32)


def _sc_body(table_hbm, x_hbm, pe_hbm, out_hbm, idx_v, pe_v, rows_v,
             gsem, ssem):
    wid = lax.axis_index("s") * NC + lax.axis_index("c")
    seq0 = wid * SEQ_PER_W
    pltpu.sync_copy(x_hbm.at[pl.ds(seq0, SEQ_PER_W)], idx_v)
    pltpu.sync_copy(pe_hbm, pe_v)

    def chunk_idx(c):
        t = lax.div(c, NCH)
        k = lax.rem(c, NCH)
        return t, k

    def fire_gather(c, b):
        t, k = chunk_idx(c)
        pltpu.async_copy(
            table_hbm.at[idx_v.at[t, pl.ds(k * CHUNK, CHUNK)]],
            rows_v.at[b], gsem)

    def wait_gather(c, b):
        t, k = chunk_idx(c)
        pltpu.make_async_copy(
            table_hbm.at[idx_v.at[t, pl.ds(k * CHUNK, CHUNK)]],
            rows_v.at[b], gsem).wait()

    def wait_scatter(b):
        pltpu.make_async_copy(
            rows_v.at[b], out_hbm.at[0, pl.ds(0, CHUNK)], ssem).wait()

    for b in range(LOOKAHEAD):
        fire_gather(b, b)

    def q_body(qi, _):
        q = qi * NBUF
        for b in range(NBUF):
            c = q + b
            ca = c + LOOKAHEAD
            ba = (b + LOOKAHEAD) % NBUF

            @pl.when(jnp.logical_and(ca >= NBUF, ca < NCHW))
            def _():
                wait_scatter(ba)
                fire_gather(ca, ba)

            @pl.when(jnp.logical_and(ca >= LOOKAHEAD, ca < NBUF))
            def _():
                fire_gather(ca, ba)

            wait_gather(c, b)
            t, k = chunk_idx(c)
            m = k * CHUNK

            def row_body(r, _):
                for j in range(D_MODEL // LANES):
                    sl = pl.ds(j * LANES, LANES)
                    rows_v[b, r, sl] = (
                        rows_v[b, r, sl] * 8.0 + pe_v[m + r, sl])
                return 0

            lax.fori_loop(0, CHUNK, row_body, 0, unroll=2)
            pltpu.async_copy(
                rows_v.at[b],
                out_hbm.at[seq0 + t, pl.ds(m, CHUNK)], ssem)
        return 0

    lax.fori_loop(0, NCHW // NBUF, q_body, 0)
    for b in range(NBUF):
        wait_scatter(b)


@jax.jit
def _run(table, x, pe):
    mesh = plsc.VectorSubcoreMesh(core_axis_name="c", subcore_axis_name="s")
    f = functools.partial(
        pl.kernel,
        mesh=mesh,
        out_type=jax.ShapeDtypeStruct((BATCH, SEQ, D_MODEL), jnp.float32),
        scratch_types=[
            pltpu.VMEM((SEQ_PER_W, SEQ), jnp.int32),
            pltpu.VMEM((SEQ, D_MODEL), jnp.float32),
            pltpu.VMEM((NBUF, CHUNK, D_MODEL), jnp.float32),
            pltpu.SemaphoreType.DMA,
            pltpu.SemaphoreType.DMA,
        ],
        compiler_params=pltpu.CompilerParams(use_tc_tiling_on_sc=False),
    )(_sc_body)
    return f(table, x, pe)


def kernel(x, table):
    pe = jnp.asarray(_pos_encoding(SEQ, D_MODEL))
    return _run(table, x.astype(jnp.int32), pe)


# speedup vs baseline: 1.2338x; 1.0729x over previous
import functools

import jax
import jax.numpy as jnp
import numpy as np
from jax import lax
from jax.experimental import pallas as pl
from jax.experimental.pallas import tpu as pltpu
from jax.experimental.pallas import tpu_sc as plsc

D_MODEL = 64
SEQ = 200
BATCH = 1024
CHUNK = 40
NCH = SEQ // CHUNK

_info = plsc.get_sparse_core_info()
NC, NS, LANES = _info.num_cores, _info.num_subcores, _info.num_lanes
NW = NC * NS
SEQ_PER_W = BATCH // NW

NBUF = 8
LOOKAHEAD = 4
NCHW = SEQ_PER_W * NCH


VOCAB = 1000000
TN = 2048


def _untile_body(in_ref, out_ref):
    xt = jnp.transpose(in_ref[...])
    xt3 = xt.reshape(TN // 2, 2, D_MODEL)
    out_ref[...] = jnp.concatenate([xt3[:, 0, :], xt3[:, 1, :]], axis=1)


def _untile_table(table_t):
    return pl.pallas_call(
        _untile_body,
        grid=(-(-VOCAB // TN),),
        in_specs=[pl.BlockSpec((D_MODEL, TN), lambda i: (0, i))],
        out_specs=pl.BlockSpec((TN // 2, 2 * D_MODEL), lambda i: (i, 0)),
        out_shape=jax.ShapeDtypeStruct((VOCAB // 2, 2 * D_MODEL), jnp.float32),
    )(table_t)


def _pos_encoding(seq_len: int, d_model: int) -> np.ndarray:
    pos = np.arange(seq_len)[:, np.newaxis].astype(np.float32)
    i = np.arange(d_model)[np.newaxis, :].astype(np.float32)
    angle_rates = 1.0 / np.power(
        10000.0, 2.0 * (np.floor(i / 2.0)) / np.float32(d_model))
    angle_rads = pos * angle_rates
    angle_rads[:, 0::2] = np.sin(angle_rads[:, 0::2])
    angle_rads[:, 1::2] = np.cos(angle_rads[:, 1::2])
    return angle_rads.astype(np.float32)


def _sc_body(table_hbm, x_hbm, pe_hbm, out_hbm, idx_v, pe_v, rows_v,
             gsem, ssem):
    wid = lax.axis_index("s") * NC + lax.axis_index("c")
    seq0 = wid * SEQ_PER_W
    pltpu.sync_copy(x_hbm.at[pl.ds(seq0, SEQ_PER_W)], idx_v)
    pltpu.sync_copy(pe_hbm, pe_v)

    def chunk_idx(c):
        t = lax.div(c, NCH)
        k = lax.rem(c, NCH)
        return t, k

    def fire_gather(c, b):
        t, k = chunk_idx(c)
        pltpu.async_copy(
            table_hbm.at[idx_v.at[t, pl.ds(k * CHUNK, CHUNK)]],
            rows_v.at[b], gsem)

    def wait_gather(c, b):
        t, k = chunk_idx(c)
        pltpu.make_async_copy(
            table_hbm.at[idx_v.at[t, pl.ds(k * CHUNK, CHUNK)]],
            rows_v.at[b], gsem).wait()

    def wait_scatter(b):
        pltpu.make_async_copy(
            rows_v.at[b], out_hbm.at[0, pl.ds(0, CHUNK)], ssem).wait()

    for b in range(LOOKAHEAD):
        fire_gather(b, b)

    def q_body(qi, _):
        q = qi * NBUF
        for b in range(NBUF):
            c = q + b
            ca = c + LOOKAHEAD
            ba = (b + LOOKAHEAD) % NBUF

            @pl.when(jnp.logical_and(ca >= NBUF, ca < NCHW))
            def _():
                wait_scatter(ba)
                fire_gather(ca, ba)

            @pl.when(jnp.logical_and(ca >= LOOKAHEAD, ca < NBUF))
            def _():
                fire_gather(ca, ba)

            wait_gather(c, b)
            t, k = chunk_idx(c)
            m = k * CHUNK

            def row_body(r, _):
                for j in range(D_MODEL // LANES):
                    sl = pl.ds(j * LANES, LANES)
                    rows_v[b, r, sl] = (
                        rows_v[b, r, sl] * 8.0 + pe_v[m + r, sl])
                return 0

            lax.fori_loop(0, CHUNK, row_body, 0, unroll=2)
            pltpu.async_copy(
                rows_v.at[b],
                out_hbm.at[seq0 + t, pl.ds(m, CHUNK)], ssem)
        return 0

    lax.fori_loop(0, NCHW // NBUF, q_body, 0)
    for b in range(NBUF):
        wait_scatter(b)


@jax.jit
def _run(table, x, pe):
    mesh = plsc.VectorSubcoreMesh(core_axis_name="c", subcore_axis_name="s")
    f = functools.partial(
        pl.kernel,
        mesh=mesh,
        out_type=jax.ShapeDtypeStruct((BATCH, SEQ, D_MODEL), jnp.float32),
        scratch_types=[
            pltpu.VMEM((SEQ_PER_W, SEQ), jnp.int32),
            pltpu.VMEM((SEQ, D_MODEL), jnp.float32),
            pltpu.VMEM((NBUF, CHUNK, D_MODEL), jnp.float32),
            pltpu.SemaphoreType.DMA,
            pltpu.SemaphoreType.DMA,
        ],
        compiler_params=pltpu.CompilerParams(use_tc_tiling_on_sc=False),
    )(_sc_body)
    return f(table, x, pe)


def kernel(x, table):
    pe = jnp.asarray(_pos_encoding(SEQ, D_MODEL))
    tbl = _untile_table(table.T).reshape(VOCAB, D_MODEL)
    return _run(tbl, x.astype(jnp.int32), pe)


# speedup vs baseline: 1.6342x; 1.3245x over previous
import functools

import jax
import jax.numpy as jnp
import numpy as np
from jax import lax
from jax.experimental import pallas as pl
from jax.experimental.pallas import tpu as pltpu
from jax.experimental.pallas import tpu_sc as plsc

D_MODEL = 64
SEQ = 200
BATCH = 1024
CHUNK = 40
NCH = SEQ // CHUNK

_info = plsc.get_sparse_core_info()
NC, NS, LANES = _info.num_cores, _info.num_subcores, _info.num_lanes
NW = NC * NS
SEQ_PER_W = BATCH // NW

NBUF = 8
LOOKAHEAD = 4
NCHW = SEQ_PER_W * NCH


VOCAB = 1000000
TN = 4096


def _untile_body(in_ref, out_ref):
    out_ref[:, 0:D_MODEL] = jnp.transpose(in_ref[...])


def _untile_table(table_t):
    return pl.pallas_call(
        _untile_body,
        grid=(-(-VOCAB // TN),),
        in_specs=[pl.BlockSpec((D_MODEL, TN), lambda i: (0, i))],
        out_specs=pl.BlockSpec((TN, 2 * D_MODEL), lambda i: (i, 0)),
        out_shape=jax.ShapeDtypeStruct((VOCAB, 2 * D_MODEL), jnp.float32),
    )(table_t)


def _pos_encoding(seq_len: int, d_model: int) -> np.ndarray:
    pos = np.arange(seq_len)[:, np.newaxis].astype(np.float32)
    i = np.arange(d_model)[np.newaxis, :].astype(np.float32)
    angle_rates = 1.0 / np.power(
        10000.0, 2.0 * (np.floor(i / 2.0)) / np.float32(d_model))
    angle_rads = pos * angle_rates
    angle_rads[:, 0::2] = np.sin(angle_rads[:, 0::2])
    angle_rads[:, 1::2] = np.cos(angle_rads[:, 1::2])
    return angle_rads.astype(np.float32)


def _sc_body(table_hbm, x_hbm, pe_hbm, out_hbm, idx_v, idx2_v, pe_v, rows_v,
             gsem, ssem):
    wid = lax.axis_index("s") * NC + lax.axis_index("c")
    seq0 = wid * SEQ_PER_W
    rows_per_w = SEQ_PER_W * SEQ
    pltpu.sync_copy(x_hbm.at[pl.ds(wid * rows_per_w, rows_per_w)], idx_v)
    pltpu.sync_copy(pe_hbm, pe_v)

    def remap_body(r, _):
        sl = pl.ds(r * LANES, LANES)
        v = idx_v[sl]
        idx2_v[sl] = v + v
        return 0

    lax.fori_loop(0, rows_per_w // LANES, remap_body, 0, unroll=4)

    def chunk_idx(c):
        t = lax.div(c, NCH)
        k = lax.rem(c, NCH)
        return t, k

    def fire_gather(c, b):
        pltpu.async_copy(
            table_hbm.at[idx2_v.at[pl.ds(c * CHUNK, CHUNK)]],
            rows_v.at[b], gsem)

    def wait_gather(c, b):
        pltpu.make_async_copy(
            table_hbm.at[idx2_v.at[pl.ds(c * CHUNK, CHUNK)]],
            rows_v.at[b], gsem).wait()

    def wait_scatter(b):
        pltpu.make_async_copy(
            rows_v.at[b], out_hbm.at[0, pl.ds(0, CHUNK)], ssem).wait()

    for b in range(LOOKAHEAD):
        fire_gather(b, b)

    def q_body(qi, _):
        q = qi * NBUF
        for b in range(NBUF):
            c = q + b
            ca = c + LOOKAHEAD
            ba = (b + LOOKAHEAD) % NBUF

            @pl.when(jnp.logical_and(ca >= NBUF, ca < NCHW))
            def _():
                wait_scatter(ba)
                fire_gather(ca, ba)

            @pl.when(jnp.logical_and(ca >= LOOKAHEAD, ca < NBUF))
            def _():
                fire_gather(ca, ba)

            wait_gather(c, b)
            t, k = chunk_idx(c)
            m = k * CHUNK

            def row_body(r, _):
                for j in range(D_MODEL // LANES):
                    sl = pl.ds(j * LANES, LANES)
                    rows_v[b, r, sl] = (
                        rows_v[b, r, sl] * 8.0 + pe_v[m + r, sl])
                return 0

            lax.fori_loop(0, CHUNK, row_body, 0, unroll=2)
            pltpu.async_copy(
                rows_v.at[b],
                out_hbm.at[seq0 + t, pl.ds(m, CHUNK)], ssem)
        return 0

    lax.fori_loop(0, NCHW // NBUF, q_body, 0)
    for b in range(NBUF):
        wait_scatter(b)


@jax.jit
def _run(table, x, pe):
    mesh = plsc.VectorSubcoreMesh(core_axis_name="c", subcore_axis_name="s")
    f = functools.partial(
        pl.kernel,
        mesh=mesh,
        out_type=jax.ShapeDtypeStruct((BATCH, SEQ, D_MODEL), jnp.float32),
        scratch_types=[
            pltpu.VMEM((SEQ_PER_W * SEQ,), jnp.int32),
            pltpu.VMEM((SEQ_PER_W * SEQ,), jnp.int32),
            pltpu.VMEM((SEQ, D_MODEL), jnp.float32),
            pltpu.VMEM((NBUF, CHUNK, D_MODEL), jnp.float32),
            pltpu.SemaphoreType.DMA,
            pltpu.SemaphoreType.DMA,
        ],
        compiler_params=pltpu.CompilerParams(use_tc_tiling_on_sc=False),
    )(_sc_body)
    return f(table, x, pe)


def kernel(x, table):
    pe = jnp.asarray(_pos_encoding(SEQ, D_MODEL))
    tbl = _untile_table(table.T).reshape(2 * VOCAB, D_MODEL)
    return _run(tbl, x.astype(jnp.int32).reshape(-1), pe)
